# baseline (speedup 1.0000x reference)
import functools

import jax
import jax.numpy as jnp
from jax import lax
from jax.experimental import pallas as pl
from jax.experimental.pallas import tpu as pltpu
from jax.experimental.pallas import tpu_sc as plsc

_LANES = 16
_NC = 2
_NS = 16
_NW = _NC * _NS

_VB = 2048


def _rowmean_body(t_ref, o_ref):
    o_ref[...] = jnp.mean(t_ref[...], axis=1)


def _rowmeans(table):
    v, _ = table.shape
    rows = 4096
    return pl.pallas_call(
        _rowmean_body,
        grid=(pl.cdiv(v, rows),),
        in_specs=[pl.BlockSpec((rows, table.shape[1]), lambda i: (i, 0))],
        out_specs=pl.BlockSpec((rows,), lambda i: (i,)),
        out_shape=jax.ShapeDtypeStruct((v,), jnp.float32),
    )(table)


@functools.lru_cache(maxsize=None)
def _make_sc_gather(n, vocab):
    per_w = n // _NW
    mesh = plsc.VectorSubcoreMesh(core_axis_name="c", subcore_axis_name="s")

    @functools.partial(
        pl.kernel,
        mesh=mesh,
        out_type=jax.ShapeDtypeStruct((n,), jnp.float32),
        compiler_params=pltpu.CompilerParams(needs_layout_passes=False),
        scratch_types=[
            pltpu.VMEM((per_w,), jnp.int32),
            pltpu.VMEM((vocab,), jnp.float32),
            pltpu.VMEM((per_w,), jnp.float32),
        ],
    )
    def gather_means(means_hbm, idx_hbm, out_hbm, idx_v, means_v, out_v):
        wid = lax.axis_index("s") * _NC + lax.axis_index("c")
        base = wid * per_w
        pltpu.sync_copy(idx_hbm.at[pl.ds(base, per_w)], idx_v)
        pltpu.sync_copy(means_hbm, means_v)

        def body(i, carry):
            ids = idx_v[pl.ds(i * _LANES, _LANES)]
            ids = jnp.clip(ids, 0, vocab - 1)
            out_v[pl.ds(i * _LANES, _LANES)] = plsc.load_gather(means_v, [ids])
            return carry

        lax.fori_loop(0, per_w // _LANES, body, 0)
        pltpu.sync_copy(out_v, out_hbm.at[pl.ds(base, per_w)])

    return gather_means


def _proj_body(x_ref, w_ref, b_ref, o_ref):
    acc = lax.dot_general(
        x_ref[...], w_ref[...],
        dimension_numbers=(((1,), (1,)), ((), ())),
        preferred_element_type=jnp.float32,
    )
    o_ref[...] = acc + b_ref[...]


def _project(x, w, b2):
    bsz, d = x.shape
    v = w.shape[0]
    return pl.pallas_call(
        _proj_body,
        grid=(pl.cdiv(v, _VB),),
        in_specs=[
            pl.BlockSpec((bsz, d), lambda j: (0, 0)),
            pl.BlockSpec((_VB, d), lambda j: (j, 0)),
            pl.BlockSpec((1, _VB), lambda j: (0, j)),
        ],
        out_specs=pl.BlockSpec((bsz, _VB), lambda j: (0, j)),
        out_shape=jax.ShapeDtypeStruct((bsz, v), jnp.float32),
    )(x, w, b2)


def kernel(context, table, W, b):
    bsz, c = context.shape
    vocab = table.shape[0]
    means = _rowmeans(table)
    idx = context.reshape(-1).astype(jnp.int32)
    x = _make_sc_gather(bsz * c, vocab)(means, idx)
    return _project(x.reshape(bsz, c), W, b.reshape(1, -1))
